# baseline (speedup 1.0000x reference)
import functools

import jax
import jax.numpy as jnp
from jax import lax
from jax.experimental import pallas as pl
from jax.experimental.pallas import tpu as pltpu
from jax.experimental.pallas import tpu_sc as plsc

NUM_DIV = 4
_LANES = 16
_G = 128
_R = 2


def _wh_body(f_ref, w_ref, n_ref, o_ref):
    f = f_ref[...]
    nm = n_ref[...]
    for d in range(NUM_DIV):
        wh = lax.dot_general(f, w_ref[d], (((1,), (1,)), ((), ())),
                             preferred_element_type=jnp.float32)
        o_ref[d] = wh * nm


def _post_body(a_ref, n_ref, o_ref):
    nm = n_ref[...]
    d_feat = a_ref.shape[-1]
    for d in range(NUM_DIV):
        o_ref[:, d * d_feat:(d + 1) * d_feat] = jnp.maximum(a_ref[d] * nm, 0.0)


def _sc_segment_sums(wh_flat, src, dst, div, zrows, n_nodes, n_edges, d_feat):
    n_tiles = 16
    ept = n_edges // n_tiles
    ec = 2000
    nchunks = ept // ec
    pcap = _G + ec + _G
    nacc = ((n_nodes // n_tiles + 8) // 8 * 8) * n_tiles
    zpt = nacc // n_tiles
    mesh = plsc.VectorSubcoreMesh(core_axis_name="c", subcore_axis_name="s")

    @functools.partial(
        pl.kernel,
        out_type=jax.ShapeDtypeStruct((NUM_DIV * nacc, d_feat), jnp.float32),
        mesh=mesh,
        compiler_params=pltpu.CompilerParams(needs_layout_passes=False),
        scratch_types=[
            pltpu.VMEM((2 * ec,), jnp.int32),
            pltpu.VMEM((2 * ec,), jnp.int32),
            pltpu.VMEM((2 * ec,), jnp.int32),
            pltpu.VMEM((pcap,), jnp.int32),
            pltpu.VMEM((_R, _G), jnp.int32),
            pltpu.VMEM((_R, _G), jnp.int32),
            pltpu.VMEM((_R * _G, d_feat), jnp.float32),
            pltpu.VMEM_SHARED((nacc, d_feat), jnp.float32),
            pltpu.SemaphoreType.DMA,
            pltpu.SemaphoreType.DMA,
            pltpu.SemaphoreType.DMA,
            pltpu.SemaphoreType.DMA,
        ],
    )
    def k(wh_hbm, src_hbm, dst_hbm, div_hbm, z_hbm, out_hbm,
          src_v, dst_v, div_v, pbuf, gstage, sstage, rowbuf, acc,
          esem, gsem, ssem, zsem):
        cid = lax.axis_index("c")
        sid = lax.axis_index("s")
        base_e = sid * ept
        r0 = sid * zpt

        def edge_start(ci, b):
            eb = base_e + ci * ec
            vb = pl.ds(b * ec, ec)
            pltpu.async_copy(src_hbm.at[pl.ds(eb, ec)], src_v.at[vb], esem)
            pltpu.async_copy(dst_hbm.at[pl.ds(eb, ec)], dst_v.at[vb], esem)
            pltpu.async_copy(div_hbm.at[pl.ds(eb, ec)], div_v.at[vb], esem)

        def edge_wait(b):
            for _ in range(3):
                pltpu.make_async_copy(src_hbm.at[pl.ds(0, ec)],
                                      src_v.at[pl.ds(b * ec, ec)], esem).wait()

        def scatter_wait():
            pltpu.make_async_copy(z_hbm.at[pl.ds(0, _G)],
                                  rowbuf.at[pl.ds(0, _G)], ssem).wait()

        def gather_wait():
            pltpu.make_async_copy(z_hbm.at[pl.ds(0, _G)],
                                  rowbuf.at[pl.ds(0, _G)], gsem).wait()

        def issue_gather(g, t):
            b = g % _R
            for kk in range(_G // _LANES):
                pk = pbuf[pl.ds(t * _G + kk * _LANES, _LANES)]
                gstage[b, pl.ds(kk * _LANES, _LANES)] = pk >> 14
                sstage[b, pl.ds(kk * _LANES, _LANES)] = pk & 16383
            pltpu.async_copy(wh_hbm.at[gstage.at[b]],
                             rowbuf.at[pl.ds(b * _G, _G)], gsem)

        def issue_scatter(g):
            b = g % _R
            pltpu.async_copy(rowbuf.at[pl.ds(b * _G, _G)],
                             acc.at[sstage.at[b]], ssem, add=True)

        def group_step(t, base_g):
            g = base_g + t

            @pl.when(g >= _R)
            def _():
                scatter_wait()

            issue_gather(g, t)

            @pl.when(g >= 1)
            def _():
                gather_wait()
                issue_scatter(g - 1)

            return base_g

        def drain_groups(count, base_g):
            navail = count // _G
            lax.fori_loop(0, navail, group_step, base_g, unroll=False)

            @pl.when(navail > 0)
            def _():
                for kk in range(_G // _LANES):
                    res = pbuf[pl.ds(navail * _G + kk * _LANES, _LANES)]
                    pbuf[pl.ds(kk * _LANES, _LANES)] = res

            return count - navail * _G, base_g + navail

        for phase in range(NUM_DIV // 2):
            d = phase * 2 + cid

            edge_start(0, 0)
            edge_start(1, 1)
            if phase > 0:
                pltpu.make_async_copy(acc.at[pl.ds(r0, zpt)],
                                      out_hbm.at[pl.ds(r0, zpt)], zsem).wait()
            pltpu.async_copy(z_hbm, acc.at[pl.ds(r0, zpt)], zsem).wait()
            plsc.subcore_barrier()

            def scan_chunk(b, count):
                def vec_body(i, cnt):
                    sv = src_v[pl.ds(b * ec + i * _LANES, _LANES)]
                    dv = dst_v[pl.ds(b * ec + i * _LANES, _LANES)]
                    gv = div_v[pl.ds(b * ec + i * _LANES, _LANES)]
                    m = gv == d
                    packed = ((d * n_nodes + sv) << 14) | dv
                    plsc.store_compressed(pbuf.at[pl.ds(cnt, _LANES)], packed,
                                          mask=m)
                    return cnt + jnp.sum(m.astype(jnp.int32))

                return lax.fori_loop(0, ec // _LANES, vec_body, count,
                                     unroll=False)

            def pair_body(q, carry):
                count, base_g = carry
                edge_wait(0)
                count = scan_chunk(0, count)
                count, base_g = drain_groups(count, base_g)

                @pl.when(q < nchunks // 2 - 1)
                def _():
                    edge_start(2 * q + 2, 0)

                edge_wait(1)
                count = scan_chunk(1, count)
                count, base_g = drain_groups(count, base_g)

                @pl.when(q < nchunks // 2 - 1)
                def _():
                    edge_start(2 * q + 3, 1)

                return count, base_g

            count, base_g = lax.fori_loop(
                0, nchunks // 2, pair_body,
                (jnp.int32(0), jnp.int32(0)), unroll=False)

            dummy = jnp.full((_LANES,), jnp.int32(n_nodes), jnp.int32)
            for kk in range(_G // _LANES):
                pbuf[pl.ds(count + kk * _LANES, _LANES)] = dummy
            count = (count + _G - 1) // _G * _G
            count, ngroups = drain_groups(count, base_g)

            @pl.when(ngroups >= 1)
            def _():
                gather_wait()
                issue_scatter(ngroups - 1)

            for kk in range(_R, 0, -1):
                @pl.when(ngroups >= kk)
                def _():
                    scatter_wait()

            plsc.subcore_barrier()

            o0 = d * nacc + r0
            pltpu.async_copy(acc.at[pl.ds(r0, zpt)], out_hbm.at[pl.ds(o0, zpt)],
                             zsem)

        pltpu.make_async_copy(acc.at[pl.ds(r0, zpt)],
                              out_hbm.at[pl.ds(r0, zpt)], zsem).wait()

    return k(wh_flat, src, dst, div, zrows), nacc


def kernel(feature, edge_index, subgraph_idx, norm, W):
    n_nodes, d_feat = feature.shape
    n_edges = edge_index.shape[1]
    src = edge_index[0]
    dst = edge_index[1]

    bn = 2000
    wh4 = pl.pallas_call(
        _wh_body,
        grid=(n_nodes // bn,),
        in_specs=[
            pl.BlockSpec((bn, d_feat), lambda i: (i, 0)),
            pl.BlockSpec((NUM_DIV, d_feat, d_feat), lambda i: (0, 0, 0)),
            pl.BlockSpec((bn, 1), lambda i: (i, 0)),
        ],
        out_specs=pl.BlockSpec((NUM_DIV, bn, d_feat), lambda i: (0, i, 0)),
        out_shape=jax.ShapeDtypeStruct((NUM_DIV, n_nodes, d_feat), jnp.float32),
    )(feature, W, norm)

    zrows = jnp.zeros((632, d_feat), jnp.float32)
    acc_flat, nacc = _sc_segment_sums(wh4.reshape(NUM_DIV * n_nodes, d_feat),
                                      src, dst, subgraph_idx, zrows,
                                      n_nodes, n_edges, d_feat)
    acc4 = acc_flat.reshape(NUM_DIV, nacc, d_feat)

    out = pl.pallas_call(
        _post_body,
        grid=(n_nodes // bn,),
        in_specs=[
            pl.BlockSpec((NUM_DIV, bn, d_feat), lambda i: (0, i, 0)),
            pl.BlockSpec((bn, 1), lambda i: (i, 0)),
        ],
        out_specs=pl.BlockSpec((bn, NUM_DIV * d_feat), lambda i: (i, 0)),
        out_shape=jax.ShapeDtypeStruct((n_nodes, NUM_DIV * d_feat),
                                       jnp.float32),
    )(acc4, norm)
    return out


# speedup vs baseline: 1.6347x; 1.6347x over previous
import functools

import jax
import jax.numpy as jnp
from jax import lax
from jax.experimental import pallas as pl
from jax.experimental.pallas import tpu as pltpu
from jax.experimental.pallas import tpu_sc as plsc

NUM_DIV = 4
_LANES = 16
_G = 32
_GD = 5
_SD = 5
_R = _GD + _SD


def _wh_body(f_ref, w_ref, n_ref, o_ref):
    f = f_ref[...]
    nm = n_ref[...]
    for d in range(NUM_DIV):
        wh = lax.dot_general(f, w_ref[d], (((1,), (1,)), ((), ())),
                             preferred_element_type=jnp.float32)
        o_ref[d] = wh * nm


def _post_body(a_ref, n_ref, o_ref):
    nm = n_ref[...]
    d_feat = a_ref.shape[-1]
    for d in range(NUM_DIV):
        o_ref[:, d * d_feat:(d + 1) * d_feat] = jnp.maximum(a_ref[d] * nm, 0.0)


def _sc_segment_sums(wh_flat, src, dst, div, zrows, n_nodes, n_edges, d_feat):
    n_tiles = 16
    ept = n_edges // n_tiles
    ec = 400
    nchunks = ept // ec
    pcap = _G + ec + _G
    nacc = ((n_nodes // n_tiles + 8) // 8 * 8) * n_tiles
    zpt = nacc // n_tiles
    mesh = plsc.VectorSubcoreMesh(core_axis_name="c", subcore_axis_name="s")

    @functools.partial(
        pl.kernel,
        out_type=jax.ShapeDtypeStruct((NUM_DIV * nacc, d_feat), jnp.float32),
        mesh=mesh,
        compiler_params=pltpu.CompilerParams(needs_layout_passes=False),
        scratch_types=[
            pltpu.VMEM((2 * ec,), jnp.int32),
            pltpu.VMEM((2 * ec,), jnp.int32),
            pltpu.VMEM((2 * ec,), jnp.int32),
            pltpu.VMEM((pcap,), jnp.int32),
            pltpu.VMEM((_R, _G), jnp.int32),
            pltpu.VMEM((_R, _G), jnp.int32),
            pltpu.VMEM((_R * _G, d_feat), jnp.float32),
            pltpu.VMEM_SHARED((nacc, d_feat), jnp.float32),
            pltpu.SemaphoreType.DMA,
            pltpu.SemaphoreType.DMA,
            pltpu.SemaphoreType.DMA,
            pltpu.SemaphoreType.DMA,
        ],
    )
    def k(wh_hbm, src_hbm, dst_hbm, div_hbm, z_hbm, out_hbm,
          src_v, dst_v, div_v, pbuf, gstage, sstage, rowbuf, acc,
          esem, gsem, ssem, zsem):
        cid = lax.axis_index("c")
        sid = lax.axis_index("s")
        base_e = sid * ept
        r0 = sid * zpt

        def edge_start(ci, b):
            eb = base_e + ci * ec
            vb = pl.ds(b * ec, ec)
            pltpu.async_copy(src_hbm.at[pl.ds(eb, ec)], src_v.at[vb], esem)
            pltpu.async_copy(dst_hbm.at[pl.ds(eb, ec)], dst_v.at[vb], esem)
            pltpu.async_copy(div_hbm.at[pl.ds(eb, ec)], div_v.at[vb], esem)

        def edge_wait(b):
            for _ in range(3):
                pltpu.make_async_copy(src_hbm.at[pl.ds(0, ec)],
                                      src_v.at[pl.ds(b * ec, ec)], esem).wait()

        def scatter_wait():
            pltpu.make_async_copy(z_hbm.at[pl.ds(0, _G)],
                                  rowbuf.at[pl.ds(0, _G)], ssem).wait()

        def gather_wait():
            pltpu.make_async_copy(z_hbm.at[pl.ds(0, _G)],
                                  rowbuf.at[pl.ds(0, _G)], gsem).wait()

        def issue_gather(g, t):
            b = g % _R
            for kk in range(_G // _LANES):
                pk = pbuf[pl.ds(t * _G + kk * _LANES, _LANES)]
                gstage[b, pl.ds(kk * _LANES, _LANES)] = pk >> 14
                sstage[b, pl.ds(kk * _LANES, _LANES)] = pk & 16383
            pltpu.async_copy(wh_hbm.at[gstage.at[b]],
                             rowbuf.at[pl.ds(b * _G, _G)], gsem)

        def issue_scatter(g):
            b = g % _R
            pltpu.async_copy(rowbuf.at[pl.ds(b * _G, _G)],
                             acc.at[sstage.at[b]], ssem, add=True)

        def group_step(t, base_g):
            g = base_g + t

            @pl.when(g >= _R)
            def _():
                scatter_wait()

            issue_gather(g, t)

            @pl.when(g >= _GD)
            def _():
                gather_wait()
                issue_scatter(g - _GD)

            return base_g

        def drain_groups(count, base_g):
            navail = count // _G
            lax.fori_loop(0, navail, group_step, base_g, unroll=False)

            @pl.when(navail > 0)
            def _():
                for kk in range(_G // _LANES):
                    res = pbuf[pl.ds(navail * _G + kk * _LANES, _LANES)]
                    pbuf[pl.ds(kk * _LANES, _LANES)] = res

            return count - navail * _G, base_g + navail

        for phase in range(NUM_DIV // 2):
            d = phase * 2 + cid

            edge_start(0, 0)
            edge_start(1, 1)
            if phase > 0:
                pltpu.make_async_copy(acc.at[pl.ds(r0, zpt)],
                                      out_hbm.at[pl.ds(r0, zpt)], zsem).wait()
            pltpu.async_copy(z_hbm, acc.at[pl.ds(r0, zpt)], zsem).wait()
            plsc.subcore_barrier()

            def scan_chunk(b, count):
                def vec_body(i, cnt):
                    sv = src_v[pl.ds(b * ec + i * _LANES, _LANES)]
                    dv = dst_v[pl.ds(b * ec + i * _LANES, _LANES)]
                    gv = div_v[pl.ds(b * ec + i * _LANES, _LANES)]
                    m = gv == d
                    packed = ((d * n_nodes + sv) << 14) | dv
                    plsc.store_compressed(pbuf.at[pl.ds(cnt, _LANES)], packed,
                                          mask=m)
                    return cnt + jnp.sum(m.astype(jnp.int32))

                return lax.fori_loop(0, ec // _LANES, vec_body, count,
                                     unroll=False)

            def pair_body(q, carry):
                count, base_g = carry
                edge_wait(0)
                count = scan_chunk(0, count)
                count, base_g = drain_groups(count, base_g)

                @pl.when(q < nchunks // 2 - 1)
                def _():
                    edge_start(2 * q + 2, 0)

                edge_wait(1)
                count = scan_chunk(1, count)
                count, base_g = drain_groups(count, base_g)

                @pl.when(q < nchunks // 2 - 1)
                def _():
                    edge_start(2 * q + 3, 1)

                return count, base_g

            count, base_g = lax.fori_loop(
                0, nchunks // 2, pair_body,
                (jnp.int32(0), jnp.int32(0)), unroll=False)

            dummy = jnp.full((_LANES,), jnp.int32(n_nodes), jnp.int32)
            for kk in range(_G // _LANES):
                pbuf[pl.ds(count + kk * _LANES, _LANES)] = dummy
            count = (count + _G - 1) // _G * _G
            count, ngroups = drain_groups(count, base_g)

            for kk in range(_GD, 0, -1):
                @pl.when(ngroups >= kk)
                def _(kk=kk):
                    gather_wait()
                    issue_scatter(ngroups - kk)

            for kk in range(_R, 0, -1):
                @pl.when(ngroups >= kk)
                def _():
                    scatter_wait()

            plsc.subcore_barrier()

            o0 = d * nacc + r0
            pltpu.async_copy(acc.at[pl.ds(r0, zpt)], out_hbm.at[pl.ds(o0, zpt)],
                             zsem)

        pltpu.make_async_copy(acc.at[pl.ds(r0, zpt)],
                              out_hbm.at[pl.ds(r0, zpt)], zsem).wait()

    return k(wh_flat, src, dst, div, zrows), nacc


def kernel(feature, edge_index, subgraph_idx, norm, W):
    n_nodes, d_feat = feature.shape
    n_edges = edge_index.shape[1]
    src = edge_index[0]
    dst = edge_index[1]

    bn = 2000
    wh4 = pl.pallas_call(
        _wh_body,
        grid=(n_nodes // bn,),
        in_specs=[
            pl.BlockSpec((bn, d_feat), lambda i: (i, 0)),
            pl.BlockSpec((NUM_DIV, d_feat, d_feat), lambda i: (0, 0, 0)),
            pl.BlockSpec((bn, 1), lambda i: (i, 0)),
        ],
        out_specs=pl.BlockSpec((NUM_DIV, bn, d_feat), lambda i: (0, i, 0)),
        out_shape=jax.ShapeDtypeStruct((NUM_DIV, n_nodes, d_feat), jnp.float32),
    )(feature, W, norm)

    zrows = jnp.zeros((632, d_feat), jnp.float32)
    acc_flat, nacc = _sc_segment_sums(wh4.reshape(NUM_DIV * n_nodes, d_feat),
                                      src, dst, subgraph_idx, zrows,
                                      n_nodes, n_edges, d_feat)
    acc4 = acc_flat.reshape(NUM_DIV, nacc, d_feat)

    out = pl.pallas_call(
        _post_body,
        grid=(n_nodes // bn,),
        in_specs=[
            pl.BlockSpec((NUM_DIV, bn, d_feat), lambda i: (0, i, 0)),
            pl.BlockSpec((bn, 1), lambda i: (i, 0)),
        ],
        out_specs=pl.BlockSpec((bn, NUM_DIV * d_feat), lambda i: (i, 0)),
        out_shape=jax.ShapeDtypeStruct((n_nodes, NUM_DIV * d_feat),
                                       jnp.float32),
    )(acc4, norm)
    return out


# speedup vs baseline: 1.8075x; 1.1057x over previous
import functools

import jax
import jax.numpy as jnp
from jax import lax
from jax.experimental import pallas as pl
from jax.experimental.pallas import tpu as pltpu
from jax.experimental.pallas import tpu_sc as plsc

NUM_DIV = 4
_LANES = 16
_G = 16
_GD = 10
_SD = 9
_R = _GD + _SD


def _wh_body(f_ref, w_ref, n_ref, o_ref):
    f = f_ref[...]
    nm = n_ref[...]
    for d in range(NUM_DIV):
        wh = lax.dot_general(f, w_ref[d], (((1,), (1,)), ((), ())),
                             preferred_element_type=jnp.float32)
        o_ref[d] = wh * nm


def _post_body(a_ref, n_ref, o_ref):
    nm = n_ref[...]
    d_feat = a_ref.shape[-1]
    for d in range(NUM_DIV):
        o_ref[:, d * d_feat:(d + 1) * d_feat] = jnp.maximum(a_ref[d] * nm, 0.0)


def _sc_segment_sums(wh_flat, src, dst, div, zrows, n_nodes, n_edges, d_feat):
    n_tiles = 16
    ept = n_edges // n_tiles
    ec = 400
    nchunks = ept // ec
    pcap = _G + ec + _G
    nacc = ((n_nodes // n_tiles + 8) // 8 * 8) * n_tiles
    zpt = nacc // n_tiles
    mesh = plsc.VectorSubcoreMesh(core_axis_name="c", subcore_axis_name="s")

    @functools.partial(
        pl.kernel,
        out_type=jax.ShapeDtypeStruct((NUM_DIV * nacc, d_feat), jnp.float32),
        mesh=mesh,
        compiler_params=pltpu.CompilerParams(needs_layout_passes=False),
        scratch_types=[
            pltpu.VMEM((2 * ec,), jnp.int32),
            pltpu.VMEM((2 * ec,), jnp.int32),
            pltpu.VMEM((2 * ec,), jnp.int32),
            pltpu.VMEM((pcap,), jnp.int32),
            pltpu.VMEM((_R, _G), jnp.int32),
            pltpu.VMEM((_R, _G), jnp.int32),
            pltpu.VMEM((_R * _G, d_feat), jnp.float32),
            pltpu.VMEM_SHARED((nacc, d_feat), jnp.float32),
            pltpu.SemaphoreType.DMA,
            pltpu.SemaphoreType.DMA,
            pltpu.SemaphoreType.DMA,
            pltpu.SemaphoreType.DMA,
        ],
    )
    def k(wh_hbm, src_hbm, dst_hbm, div_hbm, z_hbm, out_hbm,
          src_v, dst_v, div_v, pbuf, gstage, sstage, rowbuf, acc,
          esem, gsem, ssem, zsem):
        cid = lax.axis_index("c")
        sid = lax.axis_index("s")
        base_e = sid * ept
        r0 = sid * zpt

        def edge_start(ci, b):
            eb = base_e + ci * ec
            vb = pl.ds(b * ec, ec)
            pltpu.async_copy(src_hbm.at[pl.ds(eb, ec)], src_v.at[vb], esem)
            pltpu.async_copy(dst_hbm.at[pl.ds(eb, ec)], dst_v.at[vb], esem)
            pltpu.async_copy(div_hbm.at[pl.ds(eb, ec)], div_v.at[vb], esem)

        def edge_wait(b):
            for _ in range(3):
                pltpu.make_async_copy(src_hbm.at[pl.ds(0, ec)],
                                      src_v.at[pl.ds(b * ec, ec)], esem).wait()

        def scatter_wait():
            pltpu.make_async_copy(z_hbm.at[pl.ds(0, _G)],
                                  rowbuf.at[pl.ds(0, _G)], ssem).wait()

        def gather_wait():
            pltpu.make_async_copy(z_hbm.at[pl.ds(0, _G)],
                                  rowbuf.at[pl.ds(0, _G)], gsem).wait()

        def issue_gather(g, t):
            b = g % _R
            for kk in range(_G // _LANES):
                pk = pbuf[pl.ds(t * _G + kk * _LANES, _LANES)]
                gstage[b, pl.ds(kk * _LANES, _LANES)] = pk >> 14
                sstage[b, pl.ds(kk * _LANES, _LANES)] = pk & 16383
            pltpu.async_copy(wh_hbm.at[gstage.at[b]],
                             rowbuf.at[pl.ds(b * _G, _G)], gsem)

        def issue_scatter(g):
            b = g % _R
            pltpu.async_copy(rowbuf.at[pl.ds(b * _G, _G)],
                             acc.at[sstage.at[b]], ssem, add=True)

        def group_step(t, base_g):
            g = base_g + t

            @pl.when(g >= _R)
            def _():
                scatter_wait()

            issue_gather(g, t)

            @pl.when(g >= _GD)
            def _():
                gather_wait()
                issue_scatter(g - _GD)

            return base_g

        def drain_groups(count, base_g):
            navail = count // _G
            lax.fori_loop(0, navail, group_step, base_g, unroll=False)

            @pl.when(navail > 0)
            def _():
                for kk in range(_G // _LANES):
                    res = pbuf[pl.ds(navail * _G + kk * _LANES, _LANES)]
                    pbuf[pl.ds(kk * _LANES, _LANES)] = res

            return count - navail * _G, base_g + navail

        for phase in range(NUM_DIV // 2):
            d = phase * 2 + cid

            edge_start(0, 0)
            edge_start(1, 1)
            if phase > 0:
                pltpu.make_async_copy(acc.at[pl.ds(r0, zpt)],
                                      out_hbm.at[pl.ds(r0, zpt)], zsem).wait()
            pltpu.async_copy(z_hbm, acc.at[pl.ds(r0, zpt)], zsem).wait()
            plsc.subcore_barrier()

            def scan_chunk(b, count):
                def vec_body(i, cnt):
                    sv = src_v[pl.ds(b * ec + i * _LANES, _LANES)]
                    dv = dst_v[pl.ds(b * ec + i * _LANES, _LANES)]
                    gv = div_v[pl.ds(b * ec + i * _LANES, _LANES)]
                    m = gv == d
                    packed = ((d * n_nodes + sv) << 14) | dv
                    plsc.store_compressed(pbuf.at[pl.ds(cnt, _LANES)], packed,
                                          mask=m)
                    return cnt + jnp.sum(m.astype(jnp.int32))

                return lax.fori_loop(0, ec // _LANES, vec_body, count,
                                     unroll=False)

            def pair_body(q, carry):
                count, base_g = carry
                edge_wait(0)
                count = scan_chunk(0, count)
                count, base_g = drain_groups(count, base_g)

                @pl.when(q < nchunks // 2 - 1)
                def _():
                    edge_start(2 * q + 2, 0)

                edge_wait(1)
                count = scan_chunk(1, count)
                count, base_g = drain_groups(count, base_g)

                @pl.when(q < nchunks // 2 - 1)
                def _():
                    edge_start(2 * q + 3, 1)

                return count, base_g

            count, base_g = lax.fori_loop(
                0, nchunks // 2, pair_body,
                (jnp.int32(0), jnp.int32(0)), unroll=False)

            dummy = jnp.full((_LANES,), jnp.int32(n_nodes), jnp.int32)
            for kk in range(_G // _LANES):
                pbuf[pl.ds(count + kk * _LANES, _LANES)] = dummy
            count = (count + _G - 1) // _G * _G
            count, ngroups = drain_groups(count, base_g)

            for kk in range(_GD, 0, -1):
                @pl.when(ngroups >= kk)
                def _(kk=kk):
                    gather_wait()
                    issue_scatter(ngroups - kk)

            for kk in range(_R, 0, -1):
                @pl.when(ngroups >= kk)
                def _():
                    scatter_wait()

            plsc.subcore_barrier()

            o0 = d * nacc + r0
            pltpu.async_copy(acc.at[pl.ds(r0, zpt)], out_hbm.at[pl.ds(o0, zpt)],
                             zsem)

        pltpu.make_async_copy(acc.at[pl.ds(r0, zpt)],
                              out_hbm.at[pl.ds(r0, zpt)], zsem).wait()

    return k(wh_flat, src, dst, div, zrows), nacc


def kernel(feature, edge_index, subgraph_idx, norm, W):
    n_nodes, d_feat = feature.shape
    n_edges = edge_index.shape[1]
    src = edge_index[0]
    dst = edge_index[1]

    bn = 2000
    wh4 = pl.pallas_call(
        _wh_body,
        grid=(n_nodes // bn,),
        in_specs=[
            pl.BlockSpec((bn, d_feat), lambda i: (i, 0)),
            pl.BlockSpec((NUM_DIV, d_feat, d_feat), lambda i: (0, 0, 0)),
            pl.BlockSpec((bn, 1), lambda i: (i, 0)),
        ],
        out_specs=pl.BlockSpec((NUM_DIV, bn, d_feat), lambda i: (0, i, 0)),
        out_shape=jax.ShapeDtypeStruct((NUM_DIV, n_nodes, d_feat), jnp.float32),
    )(feature, W, norm)

    zrows = jnp.zeros((632, d_feat), jnp.float32)
    acc_flat, nacc = _sc_segment_sums(wh4.reshape(NUM_DIV * n_nodes, d_feat),
                                      src, dst, subgraph_idx, zrows,
                                      n_nodes, n_edges, d_feat)
    acc4 = acc_flat.reshape(NUM_DIV, nacc, d_feat)

    out = pl.pallas_call(
        _post_body,
        grid=(n_nodes // bn,),
        in_specs=[
            pl.BlockSpec((NUM_DIV, bn, d_feat), lambda i: (0, i, 0)),
            pl.BlockSpec((bn, 1), lambda i: (i, 0)),
        ],
        out_specs=pl.BlockSpec((bn, NUM_DIV * d_feat), lambda i: (i, 0)),
        out_shape=jax.ShapeDtypeStruct((n_nodes, NUM_DIV * d_feat),
                                       jnp.float32),
    )(acc4, norm)
    return out
